# baseline (speedup 1.0000x reference)
import functools

import jax
import jax.numpy as jnp
from jax.experimental import pallas as pl


def _body(gi_ref, qs_ref, qo_ref, g_ref, u_ref, o_ref, *, tb: int, d: int, g: int):
    i = pl.program_id(0)

    sizes = gi_ref[...]
    k = jax.lax.broadcasted_iota(jnp.int32, (g, g), 0)
    gg = jax.lax.broadcasted_iota(jnp.int32, (g, g), 1)
    zeros = jnp.zeros((g, g), jnp.int32)
    ends = jnp.sum(jnp.where(k <= gg, sizes, zeros), axis=0, keepdims=True)
    starts = jnp.sum(jnp.where(k < gg, sizes, zeros), axis=0, keepdims=True)

    rows = i * tb + jax.lax.broadcasted_iota(jnp.int32, (tb, 1), 0)
    onehot = (rows >= starts) & (rows < ends)

    fz = jnp.zeros((tb, g), jnp.float32)
    qs_row = jnp.sum(jnp.where(onehot, qs_ref[...], fz), axis=1, keepdims=True)
    qo_row = jnp.sum(jnp.where(onehot, qo_ref[...], fz), axis=1, keepdims=True)
    inv_qs = 1.0 / qs_row

    gate = g_ref[...]
    up = u_ref[...]
    act = up * jax.nn.sigmoid(up) * gate
    out = act * inv_qs + qo_row
    out = jnp.round(jnp.clip(out, -128.0, 127.0))
    o_ref[...] = out.astype(jnp.int8)


@jax.jit
def kernel(x_tensor, quant_scale, quant_offset, group_index):
    total, d2 = x_tensor.shape
    d = d2 // 2
    g = group_index.shape[0]
    tb = 1280
    grid = (pl.cdiv(total, tb),)

    gi = group_index.astype(jnp.int32).reshape(g, 1)
    qs = quant_scale.reshape(1, g)
    qo = quant_offset.reshape(1, g)

    return pl.pallas_call(
        functools.partial(_body, tb=tb, d=d, g=g),
        grid=grid,
        in_specs=[
            pl.BlockSpec((g, 1), lambda i: (0, 0)),
            pl.BlockSpec((1, g), lambda i: (0, 0)),
            pl.BlockSpec((1, g), lambda i: (0, 0)),
            pl.BlockSpec((tb, d), lambda i: (i, 0)),
            pl.BlockSpec((tb, d), lambda i: (i, 1)),
        ],
        out_specs=pl.BlockSpec((tb, d), lambda i: (i, 0)),
        out_shape=jax.ShapeDtypeStruct((total, d), jnp.int8),
    )(gi, qs, qo, x_tensor, x_tensor)


# speedup vs baseline: 1.0501x; 1.0501x over previous
import functools

import jax
import jax.numpy as jnp
from jax.experimental import pallas as pl


def _body(gi_ref, qs_ref, qo_ref, g_ref, u_ref, o_ref, *, tb: int, d: int, g: int):
    i = pl.program_id(0)

    sizes = gi_ref[...]
    k = jax.lax.broadcasted_iota(jnp.int32, (g, g), 0)
    gg = jax.lax.broadcasted_iota(jnp.int32, (g, g), 1)
    zeros = jnp.zeros((g, g), jnp.int32)
    ends = jnp.sum(jnp.where(k <= gg, sizes, zeros), axis=0, keepdims=True)
    starts = jnp.sum(jnp.where(k < gg, sizes, zeros), axis=0, keepdims=True)

    rows = i * tb + jax.lax.broadcasted_iota(jnp.int32, (tb, 1), 0)
    onehot = (rows >= starts) & (rows < ends)

    fz = jnp.zeros((tb, g), jnp.float32)
    qs_row = jnp.sum(jnp.where(onehot, qs_ref[...], fz), axis=1, keepdims=True)
    qo_row = jnp.sum(jnp.where(onehot, qo_ref[...], fz), axis=1, keepdims=True)
    inv_qs = 1.0 / qs_row

    gate = g_ref[...]
    up = u_ref[...]
    sig = 0.5 * jnp.tanh(0.5 * up) + 0.5
    act = up * sig * gate
    out = act * inv_qs + qo_row
    out = jnp.round(jnp.clip(out, -128.0, 127.0))
    o_ref[...] = out.astype(jnp.int8)


@jax.jit
def kernel(x_tensor, quant_scale, quant_offset, group_index):
    total, d2 = x_tensor.shape
    d = d2 // 2
    g = group_index.shape[0]
    tb = 1024
    grid = (pl.cdiv(total, tb),)

    gi = group_index.astype(jnp.int32).reshape(g, 1)
    qs = quant_scale.reshape(1, g)
    qo = quant_offset.reshape(1, g)

    return pl.pallas_call(
        functools.partial(_body, tb=tb, d=d, g=g),
        grid=grid,
        in_specs=[
            pl.BlockSpec((g, 1), lambda i: (0, 0)),
            pl.BlockSpec((1, g), lambda i: (0, 0)),
            pl.BlockSpec((1, g), lambda i: (0, 0)),
            pl.BlockSpec((tb, d), lambda i: (i, 0)),
            pl.BlockSpec((tb, d), lambda i: (i, 1)),
        ],
        out_specs=pl.BlockSpec((tb, d), lambda i: (i, 0)),
        out_shape=jax.ShapeDtypeStruct((total, d), jnp.int8),
    )(gi, qs, qo, x_tensor, x_tensor)
